# baseline (speedup 1.0000x reference)
import jax
import jax.numpy as jnp
from jax import lax
from jax.experimental import pallas as pl
from jax.experimental.pallas import tpu as pltpu
from jax.experimental.pallas import tpu_sc as plsc

_PARENT_IDX_COL = 19
_IND_KEEP = (0, 1, 2, 4, 5, 10)

_D = 64
_DP = 128

_NC = 2
_NS = 16
_NW = _NC * _NS
_L = 16

_TC_CH = 2000
_F1_CH = 2048
_SC_CH = 112
_SC_NCH = 14
_SC_NBUF = 6
_ROWS_PER_W = _SC_CH * _SC_NCH
_N_PAD = _ROWS_PER_W * _NW


def _feat1_body(data_ref, w_ref, b_ref, out_ref, pidx_ref):
    out_ref[...] = (
        jnp.dot(data_ref[...], w_ref[...], preferred_element_type=jnp.float32)
        + b_ref[...]
    )
    col = data_ref[:, _PARENT_IDX_COL:_PARENT_IDX_COL + 1]
    pidx_ref[...] = col.astype(jnp.int32).reshape(_F1_CH)


def _root_masked(pf_ref):
    rid = lax.broadcasted_iota(jnp.int32, (_TC_CH, 1), 0)
    root = (rid == 0) & (pl.program_id(0) == 0)
    return jnp.where(root, 0.0, pf_ref[...])


def _layer_body(feat_ref, pf_ref, wa_ref, wb_ref, b_ref, out_ref):
    feat = feat_ref[...]
    pf = _root_masked(pf_ref)
    acc = jnp.dot(feat, wa_ref[...], preferred_element_type=jnp.float32)
    acc = acc + jnp.dot(pf, wb_ref[...], preferred_element_type=jnp.float32)
    out_ref[...] = jnp.maximum(acc + b_ref[...] + feat, 0.0)


def _final_body(feat_ref, pf_ref, wa_ref, wb_ref, b_ref, wfc_ref,
                bfc_ref, prev_ref, out_ref):
    del prev_ref
    feat = feat_ref[...]
    pf = _root_masked(pf_ref)
    acc = jnp.dot(feat, wa_ref[...], preferred_element_type=jnp.float32)
    acc = acc + jnp.dot(pf, wb_ref[...], preferred_element_type=jnp.float32)
    f4 = jnp.maximum(acc + b_ref[...] + feat, 0.0)
    out_ref[...] = (
        jnp.dot(f4, wfc_ref[...], preferred_element_type=jnp.float32)
        + bfc_ref[...]
    )


def _row_spec(ch, d, off=0):
    return pl.BlockSpec((ch, d), lambda i, off=off: (i + off, 0))


def _full_spec(shape):
    return pl.BlockSpec(shape, lambda i: (0,) * len(shape))


def _sc_mesh():
    return plsc.VectorSubcoreMesh(
        core_axis_name="c", subcore_axis_name="s",
        num_cores=_NC, num_subcores=_NS)


def _wid():
    return lax.axis_index("s") * _NC + lax.axis_index("c")


def _gather_sc_body(n_last, table_hbm, idx_hbm, out_hbm, idx_v,
                    *bufs_and_sems):
    bufs = bufs_and_sems[:_SC_NBUF]
    gsems = bufs_and_sems[_SC_NBUF:2 * _SC_NBUF]
    ssems = bufs_and_sems[2 * _SC_NBUF:3 * _SC_NBUF]
    w = _wid()
    base = w * _ROWS_PER_W

    @pl.when(w < _NW - 1)
    def _():
        pltpu.sync_copy(idx_hbm.at[pl.ds(base, _ROWS_PER_W)], idx_v)

    @pl.when(w == _NW - 1)
    def _():
        pltpu.sync_copy(idx_hbm.at[pl.ds(base, n_last)],
                        idx_v.at[pl.ds(0, n_last)])
        zero = jnp.zeros((_L,), jnp.int32)
        for j in range((_ROWS_PER_W - n_last) // _L):
            idx_v[pl.ds(n_last + _L * j, _L)] = zero

    gath = [None] * _SC_NBUF
    scat = [None] * _SC_NBUF

    def start_gather(c):
        b = c % _SC_NBUF
        if scat[b] is not None:
            scat[b].wait()
            scat[b] = None
        gath[b] = pltpu.async_copy(
            table_hbm.at[idx_v.at[pl.ds(c * _SC_CH, _SC_CH)]],
            bufs[b], gsems[b])

    for c in range(min(_SC_NBUF - 1, _SC_NCH)):
        start_gather(c)
    for c in range(_SC_NCH):
        b = c % _SC_NBUF
        if c + _SC_NBUF - 1 < _SC_NCH:
            start_gather(c + _SC_NBUF - 1)
        gath[b].wait()
        scat[b] = pltpu.async_copy(
            bufs[b], out_hbm.at[pl.ds(base + c * _SC_CH, _SC_CH)], ssems[b])
    for s in scat:
        if s is not None:
            s.wait()


def _make_sc_gather(Ns):
    import functools as _ft
    n_last = Ns - (_NW - 1) * _ROWS_PER_W
    return pl.kernel(
        _ft.partial(_gather_sc_body, n_last),
        out_type=jax.ShapeDtypeStruct((_N_PAD, _DP), jnp.float32),
        mesh=_sc_mesh(),
        scratch_types=(
            [pltpu.VMEM((_ROWS_PER_W,), jnp.int32)]
            + [pltpu.VMEM((_SC_CH, _DP), jnp.float32)] * _SC_NBUF
            + [pltpu.SemaphoreType.DMA] * (2 * _SC_NBUF)
        ),
    )


def _pad_lanes(w):
    return jnp.concatenate(
        [w, jnp.zeros((_DP - w.shape[0], w.shape[1]), w.dtype)], axis=0)


def kernel(data, W1, b1, W2, b2, W3, b3, W4, b4, Wfc, bfc):
    Bs, Ns, Fs = data.shape
    M = Bs * Ns
    nb_ch = Ns // _TC_CH

    flat = data.reshape(M, Fs)

    w1_full = jnp.zeros((Fs, _DP), jnp.float32)
    w1_full = w1_full.at[jnp.array(_IND_KEEP), :_D].set(W1)
    b1_pad = jnp.zeros((1, _DP), jnp.float32).at[:, :_D].set(b1)

    def extra_weights(W, b):
        wa = jnp.zeros((_DP, _DP), jnp.float32).at[:_D, :_D].set(W[:_D])
        wb = jnp.zeros((_DP, _DP), jnp.float32).at[:_D, :_D].set(W[_D:])
        bp = jnp.zeros((1, _DP), jnp.float32).at[:, :_D].set(b)
        return wa, wb, bp

    gather = _make_sc_gather(Ns)
    w2 = extra_weights(W2, b2)
    w3 = extra_weights(W3, b3)
    w4 = extra_weights(W4, b4)
    out_d = Wfc.shape[1]
    wfc_pad = _pad_lanes(Wfc)
    bfc_row = bfc.reshape(1, out_d)

    def feat1_block(blk):
        n1_ch = -(-Ns // _F1_CH)
        return pl.pallas_call(
            _feat1_body,
            grid=(n1_ch,),
            in_specs=[
                _row_spec(_F1_CH, Fs),
                _full_spec((Fs, _DP)),
                _full_spec((1, _DP)),
            ],
            out_specs=[_row_spec(_F1_CH, _DP),
                       pl.BlockSpec((_F1_CH,), lambda i: (i,))],
            out_shape=[
                jax.ShapeDtypeStruct((Ns, _DP), jnp.float32),
                jax.ShapeDtypeStruct((Ns,), jnp.int32),
            ],
        )(data[blk], w1_full, b1_pad)

    def layer_block(feat, idx, weights):
        pf = gather(feat, idx)
        wa, wb, bp = weights
        return pl.pallas_call(
            _layer_body,
            grid=(nb_ch,),
            in_specs=[
                _row_spec(_TC_CH, _DP),
                _row_spec(_TC_CH, _DP),
                _full_spec((_DP, _DP)),
                _full_spec((_DP, _DP)),
                _full_spec((1, _DP)),
            ],
            out_specs=_row_spec(_TC_CH, _DP),
            out_shape=jax.ShapeDtypeStruct((Ns, _DP), jnp.float32),
        )(feat, pf, wa, wb, bp)

    out = None
    for blk in range(Bs):
        feat1, idxcol = feat1_block(blk)
        feat2 = layer_block(feat1, idxcol, w2)
        feat3 = layer_block(feat2, idxcol, w3)

        pf = gather(feat3, idxcol)
        args = [feat3, pf, *w4, wfc_pad, bfc_row]
        in_specs = [
            _row_spec(_TC_CH, _DP),
            _row_spec(_TC_CH, _DP),
            _full_spec((_DP, _DP)),
            _full_spec((_DP, _DP)),
            _full_spec((1, _DP)),
            _full_spec((_DP, out_d)),
            _full_spec((1, out_d)),
        ]
        if out is None:
            args.append(jnp.zeros((8, out_d), jnp.float32))
        else:
            args.append(out)
        in_specs.append(_full_spec((8, out_d)))
        aliases = {} if out is None else {len(args) - 1: 0}
        out = pl.pallas_call(
            _final_body,
            grid=(nb_ch,),
            in_specs=in_specs,
            out_specs=_row_spec(_TC_CH, out_d, off=blk * nb_ch),
            out_shape=jax.ShapeDtypeStruct((M, out_d), jnp.float32),
            input_output_aliases=aliases,
        )(*args)

    return out.reshape(Bs, Ns, out_d)


# speedup vs baseline: 1.0116x; 1.0116x over previous
import jax
import jax.numpy as jnp
from jax import lax
from jax.experimental import pallas as pl
from jax.experimental.pallas import tpu as pltpu
from jax.experimental.pallas import tpu_sc as plsc

_PARENT_IDX_COL = 19
_IND_KEEP = (0, 1, 2, 4, 5, 10)

_D = 64
_DP = 128

_NC = 2
_NS = 16
_NW = _NC * _NS
_L = 16

_TC_CH = 2000
_F1_CH = 2048
_SC_CH = 224
_SC_NCH = 7
_SC_NBUF = 4
_ROWS_PER_W = _SC_CH * _SC_NCH
_N_PAD = _ROWS_PER_W * _NW


def _feat1_body(data_ref, w_ref, b_ref, out_ref, pidx_ref):
    out_ref[...] = (
        jnp.dot(data_ref[...], w_ref[...], preferred_element_type=jnp.float32)
        + b_ref[...]
    )
    col = data_ref[:, _PARENT_IDX_COL:_PARENT_IDX_COL + 1]
    pidx_ref[...] = col.astype(jnp.int32).reshape(_F1_CH)


def _root_masked(pf_ref):
    rid = lax.broadcasted_iota(jnp.int32, (_TC_CH, 1), 0)
    root = (rid == 0) & (pl.program_id(0) == 0)
    return jnp.where(root, 0.0, pf_ref[...])


def _layer_body(feat_ref, pf_ref, wa_ref, wb_ref, b_ref, out_ref):
    feat = feat_ref[...]
    pf = _root_masked(pf_ref)
    acc = jnp.dot(feat, wa_ref[...], preferred_element_type=jnp.float32)
    acc = acc + jnp.dot(pf, wb_ref[...], preferred_element_type=jnp.float32)
    out_ref[...] = jnp.maximum(acc + b_ref[...] + feat, 0.0)


def _final_body(feat_ref, pf_ref, wa_ref, wb_ref, b_ref, wfc_ref,
                bfc_ref, prev_ref, out_ref):
    del prev_ref
    feat = feat_ref[...]
    pf = _root_masked(pf_ref)
    acc = jnp.dot(feat, wa_ref[...], preferred_element_type=jnp.float32)
    acc = acc + jnp.dot(pf, wb_ref[...], preferred_element_type=jnp.float32)
    f4 = jnp.maximum(acc + b_ref[...] + feat, 0.0)
    out_ref[...] = (
        jnp.dot(f4, wfc_ref[...], preferred_element_type=jnp.float32)
        + bfc_ref[...]
    )


def _row_spec(ch, d, off=0):
    return pl.BlockSpec((ch, d), lambda i, off=off: (i + off, 0))


def _full_spec(shape):
    return pl.BlockSpec(shape, lambda i: (0,) * len(shape))


def _sc_mesh():
    return plsc.VectorSubcoreMesh(
        core_axis_name="c", subcore_axis_name="s",
        num_cores=_NC, num_subcores=_NS)


def _wid():
    return lax.axis_index("s") * _NC + lax.axis_index("c")


def _gather_sc_body(n_last, tblA_hbm, idxA_hbm, tblB_hbm, idxB_hbm,
                    outA_hbm, outB_hbm, idx_v, *bufs_and_sems):
    bufs = bufs_and_sems[:_SC_NBUF]
    gsems = bufs_and_sems[_SC_NBUF:2 * _SC_NBUF]
    ssems = bufs_and_sems[2 * _SC_NBUF:3 * _SC_NBUF]
    w = _wid()
    base = w * _ROWS_PER_W
    zero = jnp.zeros((_L,), jnp.int32)

    for half, idx_hbm in enumerate((idxA_hbm, idxB_hbm)):
        off = half * _ROWS_PER_W

        @pl.when(w < _NW - 1)
        def _(idx_hbm=idx_hbm, off=off):
            pltpu.sync_copy(idx_hbm.at[pl.ds(base, _ROWS_PER_W)],
                            idx_v.at[pl.ds(off, _ROWS_PER_W)])

        @pl.when(w == _NW - 1)
        def _(idx_hbm=idx_hbm, off=off):
            pltpu.sync_copy(idx_hbm.at[pl.ds(base, n_last)],
                            idx_v.at[pl.ds(off, n_last)])
            for j in range((_ROWS_PER_W - n_last) // _L):
                idx_v[pl.ds(off + n_last + _L * j, _L)] = zero

    gath = [None] * _SC_NBUF
    scat = [None] * _SC_NBUF
    n_tot = 2 * _SC_NCH

    def tbl_of(c):
        return tblA_hbm if c < _SC_NCH else tblB_hbm

    def out_of(c):
        if c < _SC_NCH:
            return outA_hbm.at[pl.ds(base + c * _SC_CH, _SC_CH)]
        return outB_hbm.at[pl.ds(base + (c - _SC_NCH) * _SC_CH, _SC_CH)]

    def start_gather(c):
        b = c % _SC_NBUF
        if scat[b] is not None:
            scat[b].wait()
            scat[b] = None
        gath[b] = pltpu.async_copy(
            tbl_of(c).at[idx_v.at[pl.ds(c * _SC_CH, _SC_CH)]],
            bufs[b], gsems[b])

    for c in range(min(_SC_NBUF - 1, n_tot)):
        start_gather(c)
    for c in range(n_tot):
        b = c % _SC_NBUF
        if c + _SC_NBUF - 1 < n_tot:
            start_gather(c + _SC_NBUF - 1)
        gath[b].wait()
        scat[b] = pltpu.async_copy(bufs[b], out_of(c), ssems[b])
    for s in scat:
        if s is not None:
            s.wait()


def _make_sc_gather(Ns):
    import functools as _ft
    n_last = Ns - (_NW - 1) * _ROWS_PER_W
    return pl.kernel(
        _ft.partial(_gather_sc_body, n_last),
        out_type=[jax.ShapeDtypeStruct((_N_PAD, _DP), jnp.float32)] * 2,
        mesh=_sc_mesh(),
        scratch_types=(
            [pltpu.VMEM((2 * _ROWS_PER_W,), jnp.int32)]
            + [pltpu.VMEM((_SC_CH, _DP), jnp.float32)] * _SC_NBUF
            + [pltpu.SemaphoreType.DMA] * (2 * _SC_NBUF)
        ),
    )


def _pad_lanes(w):
    return jnp.concatenate(
        [w, jnp.zeros((_DP - w.shape[0], w.shape[1]), w.dtype)], axis=0)


def kernel(data, W1, b1, W2, b2, W3, b3, W4, b4, Wfc, bfc):
    Bs, Ns, Fs = data.shape
    M = Bs * Ns
    nb_ch = Ns // _TC_CH

    flat = data.reshape(M, Fs)

    w1_full = jnp.zeros((Fs, _DP), jnp.float32)
    w1_full = w1_full.at[jnp.array(_IND_KEEP), :_D].set(W1)
    b1_pad = jnp.zeros((1, _DP), jnp.float32).at[:, :_D].set(b1)

    def extra_weights(W, b):
        wa = jnp.zeros((_DP, _DP), jnp.float32).at[:_D, :_D].set(W[:_D])
        wb = jnp.zeros((_DP, _DP), jnp.float32).at[:_D, :_D].set(W[_D:])
        bp = jnp.zeros((1, _DP), jnp.float32).at[:, :_D].set(b)
        return wa, wb, bp

    gather = _make_sc_gather(Ns)
    w2 = extra_weights(W2, b2)
    w3 = extra_weights(W3, b3)
    w4 = extra_weights(W4, b4)
    out_d = Wfc.shape[1]
    wfc_pad = _pad_lanes(Wfc)
    bfc_row = bfc.reshape(1, out_d)

    def feat1_block(blk):
        n1_ch = -(-Ns // _F1_CH)
        return pl.pallas_call(
            _feat1_body,
            grid=(n1_ch,),
            in_specs=[
                _row_spec(_F1_CH, Fs),
                _full_spec((Fs, _DP)),
                _full_spec((1, _DP)),
            ],
            out_specs=[_row_spec(_F1_CH, _DP),
                       pl.BlockSpec((_F1_CH,), lambda i: (i,))],
            out_shape=[
                jax.ShapeDtypeStruct((Ns, _DP), jnp.float32),
                jax.ShapeDtypeStruct((Ns,), jnp.int32),
            ],
        )(data[blk], w1_full, b1_pad)

    def tc_layer(feat, pf, weights):
        wa, wb, bp = weights
        return pl.pallas_call(
            _layer_body,
            grid=(nb_ch,),
            in_specs=[
                _row_spec(_TC_CH, _DP),
                _row_spec(_TC_CH, _DP),
                _full_spec((_DP, _DP)),
                _full_spec((_DP, _DP)),
                _full_spec((1, _DP)),
            ],
            out_specs=_row_spec(_TC_CH, _DP),
            out_shape=jax.ShapeDtypeStruct((Ns, _DP), jnp.float32),
        )(feat, pf, wa, wb, bp)

    out = None
    for blkA, blkB in ((0, 1), (2, 3)):
        f1A, idxA = feat1_block(blkA)
        f1B, idxB = feat1_block(blkB)
        pfA, pfB = gather(f1A, idxA, f1B, idxB)
        f2A = tc_layer(f1A, pfA, w2)
        f2B = tc_layer(f1B, pfB, w2)
        pfA, pfB = gather(f2A, idxA, f2B, idxB)
        f3A = tc_layer(f2A, pfA, w3)
        f3B = tc_layer(f2B, pfB, w3)
        pfA, pfB = gather(f3A, idxA, f3B, idxB)

        for blk, feat3, pf in ((blkA, f3A, pfA), (blkB, f3B, pfB)):
            args = [feat3, pf, *w4, wfc_pad, bfc_row]
            in_specs = [
                _row_spec(_TC_CH, _DP),
                _row_spec(_TC_CH, _DP),
                _full_spec((_DP, _DP)),
                _full_spec((_DP, _DP)),
                _full_spec((1, _DP)),
                _full_spec((_DP, out_d)),
                _full_spec((1, out_d)),
            ]
            if out is None:
                args.append(jnp.zeros((8, out_d), jnp.float32))
            else:
                args.append(out)
            in_specs.append(_full_spec((8, out_d)))
            aliases = {} if out is None else {len(args) - 1: 0}
            out = pl.pallas_call(
                _final_body,
                grid=(nb_ch,),
                in_specs=in_specs,
                out_specs=_row_spec(_TC_CH, out_d, off=blk * nb_ch),
                out_shape=jax.ShapeDtypeStruct((M, out_d), jnp.float32),
                input_output_aliases=aliases,
            )(*args)

    return out.reshape(Bs, Ns, out_d)
